# baseline (speedup 1.0000x reference)
import functools

import jax
import jax.numpy as jnp
from jax import lax
from jax.experimental import pallas as pl
from jax.experimental.pallas import tpu as pltpu
from jax.experimental.pallas import tpu_sc as plsc

NC = 2
NS = 16
L = 16
EPS = 1e-5
C = 32
NBUF = 4


def _rsqrt(v):
    i = plsc.bitcast(v, jnp.int32)
    y = plsc.bitcast(jnp.int32(0x5F3759DF) - (i >> 1), jnp.float32)
    for _ in range(3):
        y = y * (1.5 - 0.5 * v * y * y)
    return y


@functools.partial(jax.jit, static_argnames=("n_tok", "seq_len", "embed"))
def _run(x_flat, sg_flat, tok_table, pos_table, seg_table, gamma, beta,
         n_tok, seq_len, embed):
    NW = NC * NS
    n_per = n_tok // NW
    n_chunks = n_per // C
    KV = embed // L
    mesh = plsc.VectorSubcoreMesh(core_axis_name="c", subcore_axis_name="s",
                                  num_cores=NC, num_subcores=NS)

    @functools.partial(
        pl.kernel,
        out_type=jax.ShapeDtypeStruct((n_tok, embed), jnp.float32),
        mesh=mesh,
        compiler_params=pltpu.CompilerParams(needs_layout_passes=False),
        scratch_types=[
            pltpu.VMEM((n_per,), jnp.int32),
            pltpu.VMEM((n_per,), jnp.int32),
            pltpu.VMEM((NBUF, C, embed), jnp.float32),
            pltpu.VMEM((NBUF, C, embed), jnp.float32),
            pltpu.VMEM((2 * seq_len, embed), jnp.float32),
            pltpu.VMEM((2, embed), jnp.float32),
            pltpu.VMEM((embed,), jnp.float32),
            pltpu.VMEM((embed,), jnp.float32),
        ] + [pltpu.SemaphoreType.DMA] * (2 * NBUF) + [
        ],
    )
    def k(x_hbm, sg_hbm, tok_hbm, pos_hbm, segt_hbm, gam_hbm, bet_hbm,
          out_hbm, idx_all, seg_all, grows, obuf, pese_v, segrow_v,
          gam_v, bet_v, *sems):
        wid = lax.axis_index("s") * NC + lax.axis_index("c")
        base = wid * n_per
        gsem = list(sems[:NBUF])
        osem = list(sems[NBUF:])

        pltpu.sync_copy(gam_hbm, gam_v)
        pltpu.sync_copy(bet_hbm, bet_v)
        pltpu.sync_copy(segt_hbm, segrow_v)
        pltpu.sync_copy(pos_hbm.at[pl.ds(0, seq_len)],
                        pese_v.at[pl.ds(0, seq_len)])
        pltpu.sync_copy(pos_hbm.at[pl.ds(0, seq_len)],
                        pese_v.at[pl.ds(seq_len, seq_len)])
        pltpu.sync_copy(x_hbm.at[pl.ds(base, n_per)], idx_all)
        pltpu.sync_copy(sg_hbm.at[pl.ds(base, n_per)], seg_all)

        seg0 = [segrow_v[0, pl.ds(k0 * L, L)] for k0 in range(KV)]
        seg1 = [segrow_v[1, pl.ds(k0 * L, L)] for k0 in range(KV)]
        gam = [gam_v[pl.ds(k0 * L, L)] for k0 in range(KV)]
        bet = [bet_v[pl.ds(k0 * L, L)] for k0 in range(KV)]

        def build(j, carry):
            for k0 in range(KV):
                sl = pl.ds(k0 * L, L)
                pese_v[j, sl] = pese_v[j, sl] + seg0[k0]
                pese_v[seq_len + j, sl] = pese_v[seq_len + j, sl] + seg1[k0]
            return carry

        lax.fori_loop(0, seq_len, build, 0)

        iota = lax.iota(jnp.int32, L)
        inv_d = jnp.float32(1.0 / embed)

        def start_gather(ci, b):
            pltpu.async_copy(tok_hbm.at[idx_all.at[pl.ds(ci * C, C)]],
                             grows.at[b], gsem[b])

        def wait_gather(ci, b):
            pltpu.make_async_copy(tok_hbm.at[idx_all.at[pl.ds(ci * C, C)]],
                                  grows.at[b], gsem[b]).wait()

        def start_out(ci, b):
            pltpu.async_copy(obuf.at[b], out_hbm.at[pl.ds(base + ci * C, C)],
                             osem[b])

        def wait_out(ci, b):
            pltpu.make_async_copy(obuf.at[b],
                                  out_hbm.at[pl.ds(base + ci * C, C)],
                                  osem[b]).wait()

        def compute(ci, b):
            tb = base + ci * C
            g_v = grows.at[b]
            o_v = obuf.at[b]

            @plsc.parallel_loop(0, C // L, 1)
            def group(gi):
                t0 = gi * L
                tvec = t0 + iota
                svec = lax.rem(tb + tvec, seq_len)
                cvec = seg_all[pl.ds(ci * C + t0, L)] * seq_len + svec
                for j in range(L):
                    t = t0 + j
                    csp = jnp.take_along_axis(
                        cvec, jnp.full((L,), j, dtype=jnp.int32), axis=0)
                    hs = []
                    for k0 in range(KV):
                        pe = plsc.load_gather(pese_v, [csp, k0 * L + iota])
                        hs.append(g_v[t, pl.ds(k0 * L, L)] + pe)
                    ss = list(hs)
                    qq = [h * h for h in hs]
                    while len(ss) > 1:
                        ss = [a + b for a, b in zip(ss[::2], ss[1::2])]
                        qq = [a + b for a, b in zip(qq[::2], qq[1::2])]
                    s, q = ss[0], qq[0]
                    for sh in (8, 4, 2, 1):
                        perm = iota ^ sh
                        s = s + jnp.take_along_axis(s, perm, axis=0)
                        q = q + jnp.take_along_axis(q, perm, axis=0)
                    mean = s * inv_d
                    var = q * inv_d - mean * mean
                    r = _rsqrt(var + EPS)
                    for k0 in range(KV):
                        o_v[t, pl.ds(k0 * L, L)] = (
                            (hs[k0] - mean) * r * gam[k0] + bet[k0])

        for b in range(NBUF):
            start_gather(b, b)

        def step(ci, b):
            wait_gather(ci, b)
            pl.when(ci >= NBUF)(lambda: wait_out(ci - NBUF, b))
            compute(ci, b)
            pl.when(ci + NBUF < n_chunks)(
                lambda: start_gather(ci + NBUF, b))
            start_out(ci, b)

        def pipeline(i, carry):
            ci = i * NBUF
            for b in range(NBUF):
                step(ci + b, b)
            return carry

        lax.fori_loop(0, n_chunks // NBUF, pipeline, 0)
        for b in range(NBUF):
            wait_out(n_chunks - NBUF + b, b)

    return k(x_flat, sg_flat, tok_table, pos_table, seg_table, gamma, beta)


def kernel(x, segment_label, tok_table, pos_table, seg_table, gamma, beta):
    b, s = x.shape
    embed = tok_table.shape[1]
    out = _run(x.reshape(-1).astype(jnp.int32),
               segment_label.reshape(-1).astype(jnp.int32),
               tok_table, pos_table, seg_table, gamma, beta,
               n_tok=b * s, seq_len=s, embed=embed)
    return out.reshape(b, s, embed)


# speedup vs baseline: 1.6277x; 1.6277x over previous
import functools

import jax
import jax.numpy as jnp
from jax import lax
from jax.experimental import pallas as pl
from jax.experimental.pallas import tpu as pltpu
from jax.experimental.pallas import tpu_sc as plsc

NC = 2
NS = 16
L = 16
EPS = 1e-5
C = 128
NBUF = 2


def _rsqrt(v):
    i = plsc.bitcast(v, jnp.int32)
    y = plsc.bitcast(jnp.int32(0x5F3759DF) - (i >> 1), jnp.float32)
    for _ in range(3):
        y = y * (1.5 - 0.5 * v * y * y)
    return y


@functools.partial(jax.jit, static_argnames=("n_tok", "seq_len", "embed"))
def _run(x_flat, sg_flat, tok_table, pos_table, seg_table, gamma, beta,
         n_tok, seq_len, embed):
    NW = NC * NS
    n_per = n_tok // NW
    n_chunks = n_per // C
    KV = embed // L
    mesh = plsc.VectorSubcoreMesh(core_axis_name="c", subcore_axis_name="s",
                                  num_cores=NC, num_subcores=NS)

    @functools.partial(
        pl.kernel,
        out_type=jax.ShapeDtypeStruct((n_tok, embed), jnp.float32),
        mesh=mesh,
        compiler_params=pltpu.CompilerParams(needs_layout_passes=False),
        scratch_types=[
            pltpu.VMEM((n_per,), jnp.int32),
            pltpu.VMEM((n_per,), jnp.int32),
            pltpu.VMEM((NBUF, C, embed), jnp.float32),
            pltpu.VMEM((NBUF, C, embed), jnp.float32),
            pltpu.VMEM((2 * seq_len, embed), jnp.float32),
            pltpu.VMEM((2, embed), jnp.float32),
            pltpu.VMEM((embed,), jnp.float32),
            pltpu.VMEM((embed,), jnp.float32),
        ] + [pltpu.SemaphoreType.DMA] * (2 * NBUF) + [
        ],
    )
    def k(x_hbm, sg_hbm, tok_hbm, pos_hbm, segt_hbm, gam_hbm, bet_hbm,
          out_hbm, idx_all, seg_all, grows, obuf, pese_v, segrow_v,
          gam_v, bet_v, *sems):
        wid = lax.axis_index("s") * NC + lax.axis_index("c")
        base = wid * n_per
        gsem = list(sems[:NBUF])
        osem = list(sems[NBUF:])

        pltpu.sync_copy(gam_hbm, gam_v)
        pltpu.sync_copy(bet_hbm, bet_v)
        pltpu.sync_copy(segt_hbm, segrow_v)
        pltpu.sync_copy(pos_hbm.at[pl.ds(0, seq_len)],
                        pese_v.at[pl.ds(0, seq_len)])
        pltpu.sync_copy(pos_hbm.at[pl.ds(0, seq_len)],
                        pese_v.at[pl.ds(seq_len, seq_len)])
        pltpu.sync_copy(x_hbm.at[pl.ds(base, n_per)], idx_all)
        pltpu.sync_copy(sg_hbm.at[pl.ds(base, n_per)], seg_all)

        seg0 = [segrow_v[0, pl.ds(k0 * L, L)] for k0 in range(KV)]
        seg1 = [segrow_v[1, pl.ds(k0 * L, L)] for k0 in range(KV)]
        gam = [gam_v[pl.ds(k0 * L, L)] for k0 in range(KV)]
        bet = [bet_v[pl.ds(k0 * L, L)] for k0 in range(KV)]

        def build(j, carry):
            for k0 in range(KV):
                sl = pl.ds(k0 * L, L)
                pese_v[j, sl] = pese_v[j, sl] + seg0[k0]
                pese_v[seq_len + j, sl] = pese_v[seq_len + j, sl] + seg1[k0]
            return carry

        lax.fori_loop(0, seq_len, build, 0)

        iota = lax.iota(jnp.int32, L)
        inv_d = jnp.float32(1.0 / embed)

        @plsc.parallel_loop(0, n_per // L, 1)
        def mkcidx(i):
            o = i * L
            sv = lax.rem(base + o + iota, seq_len)
            seg_all[pl.ds(o, L)] = seg_all[pl.ds(o, L)] * seq_len + sv

        def start_gather(ci, b):
            pltpu.async_copy(tok_hbm.at[idx_all.at[pl.ds(ci * C, C)]],
                             grows.at[b], gsem[b])

        def wait_gather(ci, b):
            pltpu.make_async_copy(tok_hbm.at[idx_all.at[pl.ds(ci * C, C)]],
                                  grows.at[b], gsem[b]).wait()

        def start_out(ci, b):
            pltpu.async_copy(obuf.at[b], out_hbm.at[pl.ds(base + ci * C, C)],
                             osem[b])

        def wait_out(ci, b):
            pltpu.make_async_copy(obuf.at[b],
                                  out_hbm.at[pl.ds(base + ci * C, C)],
                                  osem[b]).wait()

        def compute(ci, b):
            g_v = grows.at[b]
            o_v = obuf.at[b]

            @plsc.parallel_loop(0, C // L, 1)
            def group(gi):
                t0 = gi * L
                cvec = seg_all[pl.ds(ci * C + t0, L)]
                for j in range(L):
                    t = t0 + j
                    csp = jnp.take_along_axis(
                        cvec, jnp.full((L,), j, dtype=jnp.int32), axis=0)
                    hs = []
                    for k0 in range(KV):
                        pe = plsc.load_gather(pese_v, [csp, k0 * L + iota])
                        hs.append(g_v[t, pl.ds(k0 * L, L)] + pe)
                    ss = list(hs)
                    qq = [h * h for h in hs]
                    while len(ss) > 1:
                        ss = [a + b for a, b in zip(ss[::2], ss[1::2])]
                        qq = [a + b for a, b in zip(qq[::2], qq[1::2])]
                    s, q = ss[0], qq[0]
                    for sh in (8, 4, 2, 1):
                        perm = iota ^ sh
                        s = s + jnp.take_along_axis(s, perm, axis=0)
                        q = q + jnp.take_along_axis(q, perm, axis=0)
                    mean = s * inv_d
                    var = q * inv_d - mean * mean
                    r = _rsqrt(var + EPS)
                    for k0 in range(KV):
                        o_v[t, pl.ds(k0 * L, L)] = (
                            (hs[k0] - mean) * r * gam[k0] + bet[k0])

        for b in range(NBUF):
            start_gather(b, b)

        def step(ci, b):
            wait_gather(ci, b)
            pl.when(ci >= NBUF)(lambda: wait_out(ci - NBUF, b))
            compute(ci, b)
            pl.when(ci + NBUF < n_chunks)(
                lambda: start_gather(ci + NBUF, b))
            start_out(ci, b)

        def pipeline(i, carry):
            ci = i * NBUF
            for b in range(NBUF):
                step(ci + b, b)
            return carry

        lax.fori_loop(0, n_chunks // NBUF, pipeline, 0)
        for b in range(NBUF):
            wait_out(n_chunks - NBUF + b, b)

    return k(x_flat, sg_flat, tok_table, pos_table, seg_table, gamma, beta)


def kernel(x, segment_label, tok_table, pos_table, seg_table, gamma, beta):
    b, s = x.shape
    embed = tok_table.shape[1]
    out = _run(x.reshape(-1).astype(jnp.int32),
               segment_label.reshape(-1).astype(jnp.int32),
               tok_table, pos_table, seg_table, gamma, beta,
               n_tok=b * s, seq_len=s, embed=embed)
    return out.reshape(b, s, embed)


# speedup vs baseline: 3.0054x; 1.8464x over previous
import functools

import jax
import jax.numpy as jnp
from jax import lax
from jax.experimental import pallas as pl
from jax.experimental.pallas import tpu as pltpu
from jax.experimental.pallas import tpu_sc as plsc

NC = 2
NS = 16
L = 16
EPS = 1e-5
C = 128
NBUF = 2


def _rsqrt(v):
    i = plsc.bitcast(v, jnp.int32)
    y = plsc.bitcast(jnp.int32(0x5F3759DF) - (i >> 1), jnp.float32)
    for _ in range(3):
        y = y * (1.5 - 0.5 * v * y * y)
    return y


@functools.partial(jax.jit, static_argnames=("n_tok", "seq_len", "embed"))
def _run(x_flat, sg_flat, tok_table, pos_table, seg_table, gamma, beta,
         n_tok, seq_len, embed):
    NW = NC * NS
    n_per = n_tok // NW
    n_chunks = n_per // C
    KV = embed // L
    mesh = plsc.VectorSubcoreMesh(core_axis_name="c", subcore_axis_name="s",
                                  num_cores=NC, num_subcores=NS)

    @functools.partial(
        pl.kernel,
        out_type=jax.ShapeDtypeStruct((n_tok, embed), jnp.float32),
        mesh=mesh,
        compiler_params=pltpu.CompilerParams(needs_layout_passes=False),
        scratch_types=[
            pltpu.VMEM((n_per,), jnp.int32),
            pltpu.VMEM((n_per,), jnp.int32),
            pltpu.VMEM((NBUF, C, embed), jnp.float32),
            pltpu.VMEM((NBUF, C, embed), jnp.float32),
            pltpu.VMEM((2 * seq_len, embed), jnp.float32),
            pltpu.VMEM((2, embed), jnp.float32),
            pltpu.VMEM((embed,), jnp.float32),
            pltpu.VMEM((embed,), jnp.float32),
        ] + [pltpu.SemaphoreType.DMA] * (2 * NBUF) + [
        ],
    )
    def k(x_hbm, sg_hbm, tok_hbm, pos_hbm, segt_hbm, gam_hbm, bet_hbm,
          out_hbm, idx_all, seg_all, grows, obuf, pese_v, segrow_v,
          gam_v, bet_v, *sems):
        wid = lax.axis_index("s") * NC + lax.axis_index("c")
        base = wid * n_per
        gsem = list(sems[:NBUF])
        osem = list(sems[NBUF:])

        pltpu.sync_copy(gam_hbm, gam_v)
        pltpu.sync_copy(bet_hbm, bet_v)
        pltpu.sync_copy(segt_hbm, segrow_v)
        pltpu.sync_copy(pos_hbm.at[pl.ds(0, seq_len)],
                        pese_v.at[pl.ds(0, seq_len)])
        pltpu.sync_copy(pos_hbm.at[pl.ds(0, seq_len)],
                        pese_v.at[pl.ds(seq_len, seq_len)])
        pltpu.sync_copy(x_hbm.at[pl.ds(base, n_per)], idx_all)
        pltpu.sync_copy(sg_hbm.at[pl.ds(base, n_per)], seg_all)

        seg0 = [segrow_v[0, pl.ds(k0 * L, L)] for k0 in range(KV)]
        seg1 = [segrow_v[1, pl.ds(k0 * L, L)] for k0 in range(KV)]
        gam = [gam_v[pl.ds(k0 * L, L)] for k0 in range(KV)]
        bet = [bet_v[pl.ds(k0 * L, L)] for k0 in range(KV)]

        def build(j, carry):
            for k0 in range(KV):
                sl = pl.ds(k0 * L, L)
                pese_v[j, sl] = pese_v[j, sl] + seg0[k0]
                pese_v[seq_len + j, sl] = pese_v[seq_len + j, sl] + seg1[k0]
            return carry

        lax.fori_loop(0, seq_len, build, 0)

        iota = lax.iota(jnp.int32, L)
        inv_d = jnp.float32(1.0 / embed)

        @plsc.parallel_loop(0, n_per // L, 1)
        def mkcidx(i):
            o = i * L
            sv = lax.rem(base + o + iota, seq_len)
            seg_all[pl.ds(o, L)] = seg_all[pl.ds(o, L)] * seq_len + sv

        def start_gather(ci, b):
            pltpu.async_copy(tok_hbm.at[idx_all.at[pl.ds(ci * C, C)]],
                             grows.at[b], gsem[b])

        def wait_gather(ci, b):
            pltpu.make_async_copy(tok_hbm.at[idx_all.at[pl.ds(ci * C, C)]],
                                  grows.at[b], gsem[b]).wait()

        def start_out(ci, b):
            pltpu.async_copy(obuf.at[b], out_hbm.at[pl.ds(base + ci * C, C)],
                             osem[b])

        def wait_out(ci, b):
            pltpu.make_async_copy(obuf.at[b],
                                  out_hbm.at[pl.ds(base + ci * C, C)],
                                  osem[b]).wait()

        def compute(ci, b):
            g_v = grows.at[b]
            o_v = obuf.at[b]

            @plsc.parallel_loop(0, C, 1, unroll=2)
            def token(t):
                u = lax.rem(t, L)
                cvec = seg_all[pl.ds(ci * C + t - u, L)]
                csp = jnp.take_along_axis(
                    cvec, jnp.full((L,), u, dtype=jnp.int32), axis=0)
                hs = []
                for k0 in range(KV):
                    pe = plsc.load_gather(pese_v, [csp, k0 * L + iota])
                    hs.append(g_v[t, pl.ds(k0 * L, L)] + pe)
                ss = list(hs)
                qq = [h * h for h in hs]
                while len(ss) > 1:
                    ss = [a + b for a, b in zip(ss[::2], ss[1::2])]
                    qq = [a + b for a, b in zip(qq[::2], qq[1::2])]
                s, q = ss[0], qq[0]
                for sh in (8, 4, 2, 1):
                    perm = iota ^ sh
                    s = s + jnp.take_along_axis(s, perm, axis=0)
                    q = q + jnp.take_along_axis(q, perm, axis=0)
                mean = s * inv_d
                var = q * inv_d - mean * mean
                r = _rsqrt(var + EPS)
                for k0 in range(KV):
                    o_v[t, pl.ds(k0 * L, L)] = (
                        (hs[k0] - mean) * r * gam[k0] + bet[k0])

        for b in range(NBUF):
            start_gather(b, b)

        def step(ci, b):
            wait_gather(ci, b)
            pl.when(ci >= NBUF)(lambda: wait_out(ci - NBUF, b))
            compute(ci, b)
            pl.when(ci + NBUF < n_chunks)(
                lambda: start_gather(ci + NBUF, b))
            start_out(ci, b)

        def pipeline(i, carry):
            ci = i * NBUF
            for b in range(NBUF):
                step(ci + b, b)
            return carry

        lax.fori_loop(0, n_chunks // NBUF, pipeline, 0)
        for b in range(NBUF):
            wait_out(n_chunks - NBUF + b, b)

    return k(x_flat, sg_flat, tok_table, pos_table, seg_table, gamma, beta)


def kernel(x, segment_label, tok_table, pos_table, seg_table, gamma, beta):
    b, s = x.shape
    embed = tok_table.shape[1]
    out = _run(x.reshape(-1).astype(jnp.int32),
               segment_label.reshape(-1).astype(jnp.int32),
               tok_table, pos_table, seg_table, gamma, beta,
               n_tok=b * s, seq_len=s, embed=embed)
    return out.reshape(b, s, embed)


# speedup vs baseline: 3.3406x; 1.1115x over previous
import functools

import jax
import jax.numpy as jnp
from jax import lax
from jax.experimental import pallas as pl
from jax.experimental.pallas import tpu as pltpu
from jax.experimental.pallas import tpu_sc as plsc

NC = 2
NS = 16
L = 16
EPS = 1e-5
C = 128
NBUF = 2


def _rsqrt(v):
    i = plsc.bitcast(v, jnp.int32)
    y = plsc.bitcast(jnp.int32(0x5F3759DF) - (i >> 1), jnp.float32)
    for _ in range(2):
        y = y * (1.5 - 0.5 * v * y * y)
    return y


@functools.partial(jax.jit, static_argnames=("n_tok", "seq_len", "embed"))
def _run(x_flat, sg_flat, tok_table, pos_table, seg_table, gamma, beta,
         n_tok, seq_len, embed):
    NW = NC * NS
    n_per = n_tok // NW
    n_chunks = n_per // C
    KV = embed // L
    mesh = plsc.VectorSubcoreMesh(core_axis_name="c", subcore_axis_name="s",
                                  num_cores=NC, num_subcores=NS)

    @functools.partial(
        pl.kernel,
        out_type=jax.ShapeDtypeStruct((n_tok, embed), jnp.float32),
        mesh=mesh,
        compiler_params=pltpu.CompilerParams(needs_layout_passes=False),
        scratch_types=[
            pltpu.VMEM((n_per,), jnp.int32),
            pltpu.VMEM((n_per,), jnp.int32),
            pltpu.VMEM((NBUF, C, embed), jnp.float32),
            pltpu.VMEM((NBUF, C, embed), jnp.float32),
            pltpu.VMEM((2 * seq_len * embed,), jnp.float32),
            pltpu.VMEM((2, embed), jnp.float32),
            pltpu.VMEM((embed,), jnp.float32),
            pltpu.VMEM((embed,), jnp.float32),
        ] + [pltpu.SemaphoreType.DMA] * (2 * NBUF) + [
        ],
    )
    def k(x_hbm, sg_hbm, tok_hbm, pos_hbm, segt_hbm, gam_hbm, bet_hbm,
          out_hbm, idx_all, seg_all, grows, obuf, pese_v, segrow_v,
          gam_v, bet_v, *sems):
        wid = lax.axis_index("s") * NC + lax.axis_index("c")
        base = wid * n_per
        gsem = list(sems[:NBUF])
        osem = list(sems[NBUF:])

        pltpu.sync_copy(gam_hbm, gam_v)
        pltpu.sync_copy(bet_hbm, bet_v)
        pltpu.sync_copy(segt_hbm, segrow_v)
        pltpu.sync_copy(pos_hbm.at[pl.ds(0, C)], grows.at[0])
        pltpu.sync_copy(pos_hbm.at[pl.ds(C, seq_len - C)],
                        grows.at[1, pl.ds(0, seq_len - C)])
        pltpu.sync_copy(x_hbm.at[pl.ds(base, n_per)], idx_all)
        pltpu.sync_copy(sg_hbm.at[pl.ds(base, n_per)], seg_all)

        seg0 = [segrow_v[0, pl.ds(k0 * L, L)] for k0 in range(KV)]
        seg1 = [segrow_v[1, pl.ds(k0 * L, L)] for k0 in range(KV)]
        gam = [gam_v[pl.ds(k0 * L, L)] for k0 in range(KV)]
        bet = [bet_v[pl.ds(k0 * L, L)] for k0 in range(KV)]

        def fill(n_rows, src_buf, row_off):
            @plsc.parallel_loop(0, n_rows, 1)
            def fill_rows(j):
                for k0 in range(KV):
                    row = src_buf[j, pl.ds(k0 * L, L)]
                    o = (row_off + j) * embed + k0 * L
                    pese_v[pl.ds(o, L)] = row + seg0[k0]
                    o2 = (seq_len + row_off + j) * embed + k0 * L
                    pese_v[pl.ds(o2, L)] = row + seg1[k0]

        fill(C, grows.at[0], 0)
        fill(seq_len - C, grows.at[1], C)

        iota = lax.iota(jnp.int32, L)
        inv_d = jnp.float32(1.0 / embed)

        @plsc.parallel_loop(0, n_per // L, 1)
        def mkcidx(i):
            o = i * L
            sv = lax.rem(base + o + iota, seq_len)
            seg_all[pl.ds(o, L)] = (
                (seg_all[pl.ds(o, L)] * seq_len + sv) * embed)

        def start_gather(ci, b):
            pltpu.async_copy(tok_hbm.at[idx_all.at[pl.ds(ci * C, C)]],
                             grows.at[b], gsem[b])

        def wait_gather(ci, b):
            pltpu.make_async_copy(tok_hbm.at[idx_all.at[pl.ds(ci * C, C)]],
                                  grows.at[b], gsem[b]).wait()

        def start_out(ci, b):
            pltpu.async_copy(obuf.at[b], out_hbm.at[pl.ds(base + ci * C, C)],
                             osem[b])

        def wait_out(ci, b):
            pltpu.make_async_copy(obuf.at[b],
                                  out_hbm.at[pl.ds(base + ci * C, C)],
                                  osem[b]).wait()

        def compute(ci, b):
            g_v = grows.at[b]
            o_v = obuf.at[b]

            @plsc.parallel_loop(0, C, 1, unroll=2)
            def token(t):
                u = lax.rem(t, L)
                cvec = seg_all[pl.ds(ci * C + t - u, L)]
                csp = jnp.take_along_axis(
                    cvec, jnp.full((L,), u, dtype=jnp.int32), axis=0)
                hs = []
                for k0 in range(KV):
                    pe = plsc.load_gather(pese_v, [csp + (k0 * L + iota)])
                    hs.append(g_v[t, pl.ds(k0 * L, L)] + pe)
                ss = list(hs)
                qq = [h * h for h in hs]
                while len(ss) > 1:
                    ss = [a + b for a, b in zip(ss[::2], ss[1::2])]
                    qq = [a + b for a, b in zip(qq[::2], qq[1::2])]
                s, q = ss[0], qq[0]
                for sh in (8, 4, 2, 1):
                    perm = iota ^ sh
                    s = s + jnp.take_along_axis(s, perm, axis=0)
                    q = q + jnp.take_along_axis(q, perm, axis=0)
                mean = s * inv_d
                var = q * inv_d - mean * mean
                r = _rsqrt(var + EPS)
                for k0 in range(KV):
                    o_v[t, pl.ds(k0 * L, L)] = (
                        (hs[k0] - mean) * r * gam[k0] + bet[k0])

        for b in range(NBUF):
            start_gather(b, b)

        def step(ci, b):
            wait_gather(ci, b)
            pl.when(ci >= NBUF)(lambda: wait_out(ci - NBUF, b))
            compute(ci, b)
            pl.when(ci + NBUF < n_chunks)(
                lambda: start_gather(ci + NBUF, b))
            start_out(ci, b)

        def pipeline(i, carry):
            ci = i * NBUF
            for b in range(NBUF):
                step(ci + b, b)
            return carry

        lax.fori_loop(0, n_chunks // NBUF, pipeline, 0)
        for b in range(NBUF):
            wait_out(n_chunks - NBUF + b, b)

    return k(x_flat, sg_flat, tok_table, pos_table, seg_table, gamma, beta)


def kernel(x, segment_label, tok_table, pos_table, seg_table, gamma, beta):
    b, s = x.shape
    embed = tok_table.shape[1]
    out = _run(x.reshape(-1).astype(jnp.int32),
               segment_label.reshape(-1).astype(jnp.int32),
               tok_table, pos_table, seg_table, gamma, beta,
               n_tok=b * s, seq_len=s, embed=embed)
    return out.reshape(b, s, embed)


# speedup vs baseline: 3.3806x; 1.0120x over previous
import functools

import jax
import jax.numpy as jnp
from jax import lax
from jax.experimental import pallas as pl
from jax.experimental.pallas import tpu as pltpu
from jax.experimental.pallas import tpu_sc as plsc

NC = 2
NS = 16
L = 16
EPS = 1e-5
C = 128
NBUF = 2


def _rsqrt(v):
    i = plsc.bitcast(v, jnp.int32)
    y = plsc.bitcast(jnp.int32(0x5F3759DF) - (i >> 1), jnp.float32)
    for _ in range(1):
        y = y * (1.5 - 0.5 * v * y * y)
    return y


@functools.partial(jax.jit, static_argnames=("n_tok", "seq_len", "embed"))
def _run(x_flat, sg_flat, tok_table, pos_table, seg_table, gamma, beta,
         n_tok, seq_len, embed):
    NW = NC * NS
    n_per = n_tok // NW
    n_chunks = n_per // C
    KV = embed // L
    mesh = plsc.VectorSubcoreMesh(core_axis_name="c", subcore_axis_name="s",
                                  num_cores=NC, num_subcores=NS)

    @functools.partial(
        pl.kernel,
        out_type=jax.ShapeDtypeStruct((n_tok, embed), jnp.float32),
        mesh=mesh,
        compiler_params=pltpu.CompilerParams(needs_layout_passes=False),
        scratch_types=[
            pltpu.VMEM((n_per,), jnp.int32),
            pltpu.VMEM((n_per,), jnp.int32),
            pltpu.VMEM((NBUF, C, embed), jnp.float32),
            pltpu.VMEM((NBUF, C, embed), jnp.float32),
            pltpu.VMEM((2 * seq_len * embed,), jnp.float32),
            pltpu.VMEM((2, embed), jnp.float32),
            pltpu.VMEM((embed,), jnp.float32),
            pltpu.VMEM((embed,), jnp.float32),
        ] + [pltpu.SemaphoreType.DMA] * (2 * NBUF) + [
        ],
    )
    def k(x_hbm, sg_hbm, tok_hbm, pos_hbm, segt_hbm, gam_hbm, bet_hbm,
          out_hbm, idx_all, seg_all, grows, obuf, pese_v, segrow_v,
          gam_v, bet_v, *sems):
        wid = lax.axis_index("s") * NC + lax.axis_index("c")
        base = wid * n_per
        gsem = list(sems[:NBUF])
        osem = list(sems[NBUF:])

        pltpu.sync_copy(gam_hbm, gam_v)
        pltpu.sync_copy(bet_hbm, bet_v)
        pltpu.sync_copy(segt_hbm, segrow_v)
        pltpu.sync_copy(pos_hbm.at[pl.ds(0, C)], grows.at[0])
        pltpu.sync_copy(pos_hbm.at[pl.ds(C, seq_len - C)],
                        grows.at[1, pl.ds(0, seq_len - C)])
        pltpu.sync_copy(x_hbm.at[pl.ds(base, n_per)], idx_all)
        pltpu.sync_copy(sg_hbm.at[pl.ds(base, n_per)], seg_all)

        seg0 = [segrow_v[0, pl.ds(k0 * L, L)] for k0 in range(KV)]
        seg1 = [segrow_v[1, pl.ds(k0 * L, L)] for k0 in range(KV)]
        gam = [gam_v[pl.ds(k0 * L, L)] for k0 in range(KV)]
        bet = [bet_v[pl.ds(k0 * L, L)] for k0 in range(KV)]

        def fill(n_rows, src_buf, row_off):
            @plsc.parallel_loop(0, n_rows, 1)
            def fill_rows(j):
                for k0 in range(KV):
                    row = src_buf[j, pl.ds(k0 * L, L)]
                    o = (row_off + j) * embed + k0 * L
                    pese_v[pl.ds(o, L)] = row + seg0[k0]
                    o2 = (seq_len + row_off + j) * embed + k0 * L
                    pese_v[pl.ds(o2, L)] = row + seg1[k0]

        fill(C, grows.at[0], 0)
        fill(seq_len - C, grows.at[1], C)

        iota = lax.iota(jnp.int32, L)
        inv_d = jnp.float32(1.0 / embed)

        @plsc.parallel_loop(0, n_per // L, 1)
        def mkcidx(i):
            o = i * L
            sv = lax.rem(base + o + iota, seq_len)
            seg_all[pl.ds(o, L)] = (
                (seg_all[pl.ds(o, L)] * seq_len + sv) * embed)

        def start_gather(ci, b):
            pltpu.async_copy(tok_hbm.at[idx_all.at[pl.ds(ci * C, C)]],
                             grows.at[b], gsem[b])

        def wait_gather(ci, b):
            pltpu.make_async_copy(tok_hbm.at[idx_all.at[pl.ds(ci * C, C)]],
                                  grows.at[b], gsem[b]).wait()

        def start_out(ci, b):
            pltpu.async_copy(obuf.at[b], out_hbm.at[pl.ds(base + ci * C, C)],
                             osem[b])

        def wait_out(ci, b):
            pltpu.make_async_copy(obuf.at[b],
                                  out_hbm.at[pl.ds(base + ci * C, C)],
                                  osem[b]).wait()

        def compute(ci, b):
            g_v = grows.at[b]
            o_v = obuf.at[b]

            @plsc.parallel_loop(0, C, 1, unroll=2)
            def token(t):
                u = lax.rem(t, L)
                cvec = seg_all[pl.ds(ci * C + t - u, L)]
                csp = jnp.take_along_axis(
                    cvec, jnp.full((L,), u, dtype=jnp.int32), axis=0)
                hs = []
                for k0 in range(KV):
                    pe = plsc.load_gather(pese_v, [csp + (k0 * L + iota)])
                    hs.append(g_v[t, pl.ds(k0 * L, L)] + pe)
                ss = list(hs)
                qq = [h * h for h in hs]
                while len(ss) > 1:
                    ss = [a + b for a, b in zip(ss[::2], ss[1::2])]
                    qq = [a + b for a, b in zip(qq[::2], qq[1::2])]
                s, q = ss[0], qq[0]
                for sh in (8, 4, 2, 1):
                    perm = iota ^ sh
                    s = s + jnp.take_along_axis(s, perm, axis=0)
                    q = q + jnp.take_along_axis(q, perm, axis=0)
                mean = s * inv_d
                var = q * inv_d - mean * mean
                r = _rsqrt(var + EPS)
                for k0 in range(KV):
                    o_v[t, pl.ds(k0 * L, L)] = (
                        (hs[k0] - mean) * r * gam[k0] + bet[k0])

        for b in range(NBUF):
            start_gather(b, b)

        def step(ci, b):
            wait_gather(ci, b)
            pl.when(ci >= NBUF)(lambda: wait_out(ci - NBUF, b))
            compute(ci, b)
            pl.when(ci + NBUF < n_chunks)(
                lambda: start_gather(ci + NBUF, b))
            start_out(ci, b)

        def pipeline(i, carry):
            ci = i * NBUF
            for b in range(NBUF):
                step(ci + b, b)
            return carry

        lax.fori_loop(0, n_chunks // NBUF, pipeline, 0)
        for b in range(NBUF):
            wait_out(n_chunks - NBUF + b, b)

    return k(x_flat, sg_flat, tok_table, pos_table, seg_table, gamma, beta)


def kernel(x, segment_label, tok_table, pos_table, seg_table, gamma, beta):
    b, s = x.shape
    embed = tok_table.shape[1]
    out = _run(x.reshape(-1).astype(jnp.int32),
               segment_label.reshape(-1).astype(jnp.int32),
               tok_table, pos_table, seg_table, gamma, beta,
               n_tok=b * s, seq_len=s, embed=embed)
    return out.reshape(b, s, embed)
